# baseline (speedup 1.0000x reference)
import functools

import jax
import jax.numpy as jnp
from jax import lax
from jax.experimental import pallas as pl
from jax.experimental.pallas import tpu as pltpu
from jax.experimental.pallas import tpu_sc as plsc

_B = 16384
_L = 20
_EMB = 32
_NC = 2
_NS = 16
_NW = _NC * _NS
_BPW = _B // _NW
_CH = 64
_NCH = _BPW // _CH
_PAIRS = _NCH // 2
_TPW = _BPW * _L


@functools.partial(
    pl.kernel,
    out_type=jax.ShapeDtypeStruct((_B // 2, 128), jnp.float32),
    scratch_types=[
        pltpu.VMEM((_TPW // 128, 128), jnp.int32),
        pltpu.VMEM((_L, _CH), jnp.int32),
        pltpu.VMEM((_L, _CH), jnp.int32),
        pltpu.VMEM((_CH, _EMB), jnp.float32),
        pltpu.VMEM((_CH, _EMB), jnp.float32),
        pltpu.VMEM((_CH // 2, 128), jnp.float32),
        pltpu.VMEM((1, _EMB), jnp.float32),
        pltpu.VMEM_SHARED((10000, _EMB), jnp.float32),
        pltpu.SemaphoreType.DMA,
        pltpu.SemaphoreType.DMA,
    ],
    mesh=plsc.VectorSubcoreMesh(core_axis_name="c", subcore_axis_name="s"),
    compiler_params=pltpu.CompilerParams(use_tc_tiling_on_sc=False,
                                         needs_layout_passes=False),
)
def _sc_item_model(tok_hbm, ttx_hbm, out_hbm,
                   tokraw_v, t0_v, t1_v, g0_v, g1_v,
                   outc_v, row0_v, ttx_sh, sem_a, sem_b):
    c = lax.axis_index("c")
    s = lax.axis_index("s")
    w = s * _NC + c
    base = w * _BPW

    pltpu.sync_copy(ttx_hbm.at[pl.ds(0, 1), :], row0_v)
    pltpu.sync_copy(tok_hbm.at[pl.ds(w * (_TPW // 128), _TPW // 128), :],
                    tokraw_v)

    iota_l = lax.iota(jnp.int32, 16) * _L

    def build(cc, tbuf):
        for l in range(_L):
            for g in range(_CH // 16):
                off = (cc * _CH + g * 16) * _L + l
                f = iota_l + jnp.full((16,), off, jnp.int32)
                fr = lax.shift_right_logical(f, 7)
                fc = lax.bitwise_and(f, 127)
                tbuf[l, pl.ds(g * 16, 16)] = plsc.load_gather(tokraw_v, [fr, fc])

    zvec = jnp.zeros((16,), jnp.float32)

    def zero(gbuf):
        def zb(r, carry):
            gbuf[r, pl.ds(0, 16)] = zvec
            gbuf[r, pl.ds(16, 16)] = zvec
            return carry
        lax.fori_loop(0, _CH, zb, 0)

    def fire(tbuf, gbuf, sem):
        for l in range(_L):
            pltpu.async_copy(ttx_sh.at[tbuf.at[l]], gbuf, sem, add=True)

    def drain(gbuf, sem):
        for _ in range(_L):
            pltpu.make_async_copy(ttx_hbm.at[pl.ds(0, _CH), :], gbuf, sem).wait()

    pltpu.sync_copy(ttx_hbm.at[pl.ds(s * 625, 625), :],
                    ttx_sh.at[pl.ds(s * 625, 625), :])
    build(0, t0_v)
    zero(g0_v)
    plsc.subcore_barrier()
    fire(t0_v, g0_v, sem_a)

    row0_lo = row0_v[0, pl.ds(0, 16)]
    row0_hi = row0_v[0, pl.ds(16, 16)]

    def compute(cc, gbuf, tbuf):
        def group_body(g, carry):
            cnt = jnp.zeros((16,), jnp.int32)
            for l in range(_L):
                t = tbuf[l, pl.ds(g * 16, 16)]
                cnt = cnt + jnp.minimum(t, 1)
            cntf = cnt.astype(jnp.float32)
            rec_vec = 1.0 / jnp.maximum(cntf, 1.0)
            pad_vec = jnp.float32(_L) - cntf

            for j in range(16):
                b = g * 16 + j
                r = cc * _CH + b
                orow = g * 8 + j // 2
                ocol = 64 * (j % 2)
                rec = jnp.full((16,), rec_vec[j], jnp.float32)
                pad = jnp.full((16,), pad_vec[j], jnp.float32)
                tlo = gbuf[b, pl.ds(0, 16)]
                thi = gbuf[b, pl.ds(16, 16)]
                outc_v[orow, pl.ds(ocol + 32, 16)] = (tlo - pad * row0_lo) * rec
                outc_v[orow, pl.ds(ocol + 48, 16)] = (thi - pad * row0_hi) * rec
            return carry

        lax.fori_loop(0, _CH // 16, group_body, 0)
        pltpu.sync_copy(outc_v,
                        out_hbm.at[pl.ds((base + cc * _CH) // 2, _CH // 2), :])

    def pair_body(i, carry):
        cc0 = 2 * i
        build(cc0 + 1, t1_v)
        zero(g1_v)
        fire(t1_v, g1_v, sem_b)
        drain(g0_v, sem_a)
        compute(cc0, g0_v, t0_v)

        @pl.when(i < _PAIRS - 1)
        def _():
            build(cc0 + 2, t0_v)
            zero(g0_v)
            fire(t0_v, g0_v, sem_a)

        drain(g1_v, sem_b)
        compute(cc0 + 1, g1_v, t1_v)
        return carry

    lax.fori_loop(0, _PAIRS, pair_body, 0)


@functools.partial(
    pl.kernel,
    out_type=jax.ShapeDtypeStruct((_B, 2 * _EMB), jnp.float32),
    scratch_types=[
        pltpu.VMEM((_BPW,), jnp.int32),
        pltpu.VMEM((_BPW, _EMB), jnp.float32),
        pltpu.VMEM((_BPW // 2, 128), jnp.float32),
        pltpu.VMEM((_BPW, 2 * _EMB), jnp.float32),
        pltpu.SemaphoreType.DMA,
    ],
    mesh=plsc.VectorSubcoreMesh(core_axis_name="c", subcore_axis_name="s"),
    compiler_params=pltpu.CompilerParams(use_tc_tiling_on_sc=False,
                                         needs_layout_passes=False),
)
def _sc_id_merge(ids_hbm, tid_hbm, outa_hbm, out_hbm,
                 ids_v, idrows_v, outaw_v, outw_v, sem_id):
    c = lax.axis_index("c")
    s = lax.axis_index("s")
    w = s * _NC + c

    pltpu.sync_copy(ids_hbm.at[pl.ds(w * _BPW, _BPW)], ids_v)
    for q in range(4):
        pltpu.async_copy(tid_hbm.at[ids_v.at[pl.ds(q * 128, 128)]],
                         idrows_v.at[pl.ds(q * 128, 128), :], sem_id)
    pltpu.sync_copy(outa_hbm.at[pl.ds(w * (_BPW // 2), _BPW // 2), :], outaw_v)
    pltpu.make_async_copy(tid_hbm.at[pl.ds(0, _BPW), :], idrows_v, sem_id).wait()

    def group_body(g, carry):
        for j in range(16):
            b = g * 16 + j
            arow = g * 8 + j // 2
            acol = 64 * (j % 2)
            outw_v[b, pl.ds(0, 16)] = idrows_v[b, pl.ds(0, 16)]
            outw_v[b, pl.ds(16, 16)] = idrows_v[b, pl.ds(16, 16)]
            outw_v[b, pl.ds(32, 16)] = outaw_v[arow, pl.ds(acol + 32, 16)]
            outw_v[b, pl.ds(48, 16)] = outaw_v[arow, pl.ds(acol + 48, 16)]
        return carry

    lax.fori_loop(0, _BPW // 16, group_body, 0)
    pltpu.sync_copy(outw_v, out_hbm.at[pl.ds(w * _BPW, _BPW), :])


@jax.jit
def kernel(item_ids, title_tokens, table_id, table_text):
    ids = item_ids.astype(jnp.int32)
    tok = title_tokens.astype(jnp.int32).reshape(_B * _L // 128, 128)
    outa = _sc_item_model(tok, table_text)
    out = _sc_id_merge(ids, table_id, outa)
    return out


# speedup vs baseline: 1.0219x; 1.0219x over previous
import functools

import jax
import jax.numpy as jnp
from jax import lax
from jax.experimental import pallas as pl
from jax.experimental.pallas import tpu as pltpu
from jax.experimental.pallas import tpu_sc as plsc

_B = 16384
_L = 20
_EMB = 32
_NC = 2
_NS = 16
_NW = _NC * _NS
_BPW = _B // _NW
_CH = 64
_NCH = _BPW // _CH
_PAIRS = _NCH // 2
_TPW = _BPW * _L


@functools.partial(
    pl.kernel,
    out_type=jax.ShapeDtypeStruct((_B // 2, 128), jnp.float32),
    scratch_types=[
        pltpu.VMEM((_TPW // 128, 128), jnp.int32),
        pltpu.VMEM((_L, _CH), jnp.int32),
        pltpu.VMEM((_L, _CH), jnp.int32),
        pltpu.VMEM((_CH, _EMB), jnp.float32),
        pltpu.VMEM((_CH, _EMB), jnp.float32),
        pltpu.VMEM((_CH // 2, 128), jnp.float32),
        pltpu.VMEM((1, _EMB), jnp.float32),
        pltpu.VMEM_SHARED((10000, _EMB), jnp.float32),
        pltpu.SemaphoreType.DMA,
        pltpu.SemaphoreType.DMA,
    ],
    mesh=plsc.VectorSubcoreMesh(core_axis_name="c", subcore_axis_name="s"),
    compiler_params=pltpu.CompilerParams(use_tc_tiling_on_sc=False,
                                         needs_layout_passes=False),
)
def _sc_item_model(tok_hbm, ttx_hbm, out_hbm,
                   tokraw_v, t0_v, t1_v, g0_v, g1_v,
                   outc_v, row0_v, ttx_sh, sem_a, sem_b):
    c = lax.axis_index("c")
    s = lax.axis_index("s")
    w = s * _NC + c
    base = w * _BPW

    pltpu.sync_copy(ttx_hbm.at[pl.ds(0, 1), :], row0_v)
    pltpu.sync_copy(tok_hbm.at[pl.ds(w * (_TPW // 128), _TPW // 128), :],
                    tokraw_v)

    iota_l = lax.iota(jnp.int32, 16) * _L

    def build(cc, tbuf):
        for l in range(_L):
            for g in range(_CH // 16):
                off = (cc * _CH + g * 16) * _L + l
                f = iota_l + jnp.full((16,), off, jnp.int32)
                fr = lax.shift_right_logical(f, 7)
                fc = lax.bitwise_and(f, 127)
                tbuf[l, pl.ds(g * 16, 16)] = plsc.load_gather(tokraw_v, [fr, fc])

    zvec = jnp.zeros((16,), jnp.float32)

    def zero(gbuf):
        def zb(r, carry):
            gbuf[r, pl.ds(0, 16)] = zvec
            gbuf[r, pl.ds(16, 16)] = zvec
            return carry
        lax.fori_loop(0, _CH, zb, 0)

    def fire(tbuf, gbuf, sem):
        for l in range(_L):
            pltpu.async_copy(ttx_sh.at[tbuf.at[l]], gbuf, sem, add=True)

    def drain(gbuf, sem):
        for _ in range(_L):
            pltpu.make_async_copy(ttx_hbm.at[pl.ds(0, _CH), :], gbuf, sem).wait()

    pltpu.sync_copy(ttx_hbm.at[pl.ds(s * 625, 625), :],
                    ttx_sh.at[pl.ds(s * 625, 625), :])
    build(0, t0_v)
    zero(g0_v)
    plsc.subcore_barrier()
    fire(t0_v, g0_v, sem_a)

    row0_lo = row0_v[0, pl.ds(0, 16)]
    row0_hi = row0_v[0, pl.ds(16, 16)]

    def compute(cc, gbuf, tbuf):
        def group_body(g, carry):
            cnt = jnp.zeros((16,), jnp.int32)
            for l in range(_L):
                t = tbuf[l, pl.ds(g * 16, 16)]
                cnt = cnt + jnp.minimum(t, 1)
            cntf = cnt.astype(jnp.float32)
            rec_vec = 1.0 / jnp.maximum(cntf, 1.0)
            pad_vec = jnp.float32(_L) - cntf

            for j in range(16):
                b = g * 16 + j
                r = cc * _CH + b
                orow = g * 8 + j // 2
                ocol = 64 * (j % 2)
                rec = jnp.full((16,), rec_vec[j], jnp.float32)
                pad = jnp.full((16,), pad_vec[j], jnp.float32)
                tlo = gbuf[b, pl.ds(0, 16)]
                thi = gbuf[b, pl.ds(16, 16)]
                outc_v[orow, pl.ds(ocol + 32, 16)] = (tlo - pad * row0_lo) * rec
                outc_v[orow, pl.ds(ocol + 48, 16)] = (thi - pad * row0_hi) * rec
            return carry

        lax.fori_loop(0, _CH // 16, group_body, 0)
        pltpu.sync_copy(outc_v,
                        out_hbm.at[pl.ds((base + cc * _CH) // 2, _CH // 2), :])

    def pair_body(i, carry):
        cc0 = 2 * i
        build(cc0 + 1, t1_v)
        zero(g1_v)
        fire(t1_v, g1_v, sem_b)
        drain(g0_v, sem_a)
        compute(cc0, g0_v, t0_v)

        @pl.when(i < _PAIRS - 1)
        def _():
            build(cc0 + 2, t0_v)
            zero(g0_v)
            fire(t0_v, g0_v, sem_a)

        drain(g1_v, sem_b)
        compute(cc0 + 1, g1_v, t1_v)
        return carry

    lax.fori_loop(0, _PAIRS, pair_body, 0)


@functools.partial(
    pl.kernel,
    out_type=jax.ShapeDtypeStruct((_B, 128), jnp.float32),
    scratch_types=[
        pltpu.VMEM((_BPW,), jnp.int32),
        pltpu.VMEM((_BPW, _EMB), jnp.float32),
        pltpu.VMEM((_BPW // 2, 128), jnp.float32),
        pltpu.VMEM((_BPW, 128), jnp.float32),
        pltpu.SemaphoreType.DMA,
    ],
    mesh=plsc.VectorSubcoreMesh(core_axis_name="c", subcore_axis_name="s"),
    compiler_params=pltpu.CompilerParams(use_tc_tiling_on_sc=False,
                                         needs_layout_passes=False),
)
def _sc_id_merge(ids_hbm, tid_hbm, outa_hbm, out_hbm,
                 ids_v, idrows_v, outaw_v, outw_v, sem_id):
    c = lax.axis_index("c")
    s = lax.axis_index("s")
    w = s * _NC + c

    pltpu.sync_copy(ids_hbm.at[pl.ds(w * _BPW, _BPW)], ids_v)
    for q in range(4):
        pltpu.async_copy(tid_hbm.at[ids_v.at[pl.ds(q * 128, 128)]],
                         idrows_v.at[pl.ds(q * 128, 128), :], sem_id)
    pltpu.sync_copy(outa_hbm.at[pl.ds(w * (_BPW // 2), _BPW // 2), :], outaw_v)
    pltpu.make_async_copy(tid_hbm.at[pl.ds(0, _BPW), :], idrows_v, sem_id).wait()

    def group_body(g, carry):
        for j in range(16):
            b = g * 16 + j
            arow = g * 8 + j // 2
            acol = 64 * (j % 2)
            outw_v[b, pl.ds(0, 16)] = idrows_v[b, pl.ds(0, 16)]
            outw_v[b, pl.ds(16, 16)] = idrows_v[b, pl.ds(16, 16)]
            outw_v[b, pl.ds(32, 16)] = outaw_v[arow, pl.ds(acol + 32, 16)]
            outw_v[b, pl.ds(48, 16)] = outaw_v[arow, pl.ds(acol + 48, 16)]
        return carry

    lax.fori_loop(0, _BPW // 16, group_body, 0)
    pltpu.sync_copy(outw_v, out_hbm.at[pl.ds(w * _BPW, _BPW), :])


@jax.jit
def kernel(item_ids, title_tokens, table_id, table_text):
    ids = item_ids.astype(jnp.int32)
    tok = title_tokens.astype(jnp.int32).reshape(_B * _L // 128, 128)
    outa = _sc_item_model(tok, table_text)
    out = _sc_id_merge(ids, table_id, outa)
    return out[:, :2 * _EMB]


# speedup vs baseline: 1.0717x; 1.0488x over previous
import functools

import jax
import jax.numpy as jnp
from jax import lax
from jax.experimental import pallas as pl
from jax.experimental.pallas import tpu as pltpu
from jax.experimental.pallas import tpu_sc as plsc

_B = 16384
_L = 20
_EMB = 32
_NC = 2
_NS = 16
_NW = _NC * _NS
_BPW = _B // _NW
_CH = 64
_NCH = _BPW // _CH
_PAIRS = _NCH // 2
_TPW = _BPW * _L


@functools.partial(
    pl.kernel,
    out_type=jax.ShapeDtypeStruct((_B // 2, 128), jnp.float32),
    scratch_types=[
        pltpu.VMEM((_TPW // 128, 128), jnp.int32),
        pltpu.VMEM((_L, _CH), jnp.int32),
        pltpu.VMEM((_L, _CH), jnp.int32),
        pltpu.VMEM((_CH, _EMB), jnp.float32),
        pltpu.VMEM((_CH, _EMB), jnp.float32),
        pltpu.VMEM((_CH // 2, 128), jnp.float32),
        pltpu.VMEM((1, _EMB), jnp.float32),
        pltpu.VMEM_SHARED((10000, _EMB), jnp.float32),
        pltpu.SemaphoreType.DMA,
        pltpu.SemaphoreType.DMA,
    ],
    mesh=plsc.VectorSubcoreMesh(core_axis_name="c", subcore_axis_name="s"),
    compiler_params=pltpu.CompilerParams(use_tc_tiling_on_sc=False,
                                         needs_layout_passes=False),
)
def _sc_item_model(tok_hbm, ttx_hbm, out_hbm,
                   tokraw_v, t0_v, t1_v, g0_v, g1_v,
                   outc_v, row0_v, ttx_sh, sem_a, sem_b):
    c = lax.axis_index("c")
    s = lax.axis_index("s")
    w = s * _NC + c
    base = w * _BPW

    pltpu.sync_copy(ttx_hbm.at[pl.ds(0, 1), :], row0_v)
    pltpu.sync_copy(tok_hbm.at[pl.ds(w * (_TPW // 128), _TPW // 128), :],
                    tokraw_v)

    iota_l = lax.iota(jnp.int32, 16) * _L

    def build(cc, tbuf):
        for l in range(_L):
            for g in range(_CH // 16):
                off = (cc * _CH + g * 16) * _L + l
                f = iota_l + jnp.full((16,), off, jnp.int32)
                fr = lax.shift_right_logical(f, 7)
                fc = lax.bitwise_and(f, 127)
                tbuf[l, pl.ds(g * 16, 16)] = plsc.load_gather(tokraw_v, [fr, fc])

    zvec = jnp.zeros((16,), jnp.float32)

    def zero(gbuf):
        def zb(r, carry):
            gbuf[r, pl.ds(0, 16)] = zvec
            gbuf[r, pl.ds(16, 16)] = zvec
            return carry
        lax.fori_loop(0, _CH, zb, 0)

    def fire(tbuf, gbuf, sem):
        for l in range(_L):
            pltpu.async_copy(ttx_sh.at[tbuf.at[l]], gbuf, sem, add=True)

    def drain(gbuf, sem):
        for _ in range(_L):
            pltpu.make_async_copy(ttx_hbm.at[pl.ds(0, _CH), :], gbuf, sem).wait()

    pltpu.sync_copy(ttx_hbm.at[pl.ds(s * 625, 625), :],
                    ttx_sh.at[pl.ds(s * 625, 625), :])
    build(0, t0_v)
    zero(g0_v)
    plsc.subcore_barrier()
    fire(t0_v, g0_v, sem_a)

    row0_lo = row0_v[0, pl.ds(0, 16)]
    row0_hi = row0_v[0, pl.ds(16, 16)]

    def compute(cc, gbuf, tbuf):
        def group_body(g, carry):
            cnt = jnp.zeros((16,), jnp.int32)
            for l in range(_L):
                t = tbuf[l, pl.ds(g * 16, 16)]
                cnt = cnt + jnp.minimum(t, 1)
            cntf = cnt.astype(jnp.float32)
            rec_vec = 1.0 / jnp.maximum(cntf, 1.0)
            pad_vec = jnp.float32(_L) - cntf

            for j in range(16):
                b = g * 16 + j
                r = cc * _CH + b
                orow = g * 8 + j // 2
                ocol = 64 * (j % 2)
                rec = jnp.full((16,), rec_vec[j], jnp.float32)
                pad = jnp.full((16,), pad_vec[j], jnp.float32)
                tlo = gbuf[b, pl.ds(0, 16)]
                thi = gbuf[b, pl.ds(16, 16)]
                outc_v[orow, pl.ds(ocol + 32, 16)] = (tlo - pad * row0_lo) * rec
                outc_v[orow, pl.ds(ocol + 48, 16)] = (thi - pad * row0_hi) * rec
            return carry

        lax.fori_loop(0, _CH // 16, group_body, 0)
        pltpu.sync_copy(outc_v,
                        out_hbm.at[pl.ds((base + cc * _CH) // 2, _CH // 2), :])

    def pair_body(i, carry):
        cc0 = 2 * i
        build(cc0 + 1, t1_v)
        zero(g1_v)
        fire(t1_v, g1_v, sem_b)
        drain(g0_v, sem_a)
        compute(cc0, g0_v, t0_v)

        @pl.when(i < _PAIRS - 1)
        def _():
            build(cc0 + 2, t0_v)
            zero(g0_v)
            fire(t0_v, g0_v, sem_a)

        drain(g1_v, sem_b)
        compute(cc0 + 1, g1_v, t1_v)
        return carry

    lax.fori_loop(0, _PAIRS, pair_body, 0)


@functools.partial(
    pl.kernel,
    out_type=jax.ShapeDtypeStruct((_B, 128), jnp.float32),
    scratch_types=[
        pltpu.VMEM((_BPW,), jnp.int32),
        pltpu.VMEM((_BPW, _EMB), jnp.float32),
        pltpu.VMEM((_BPW // 2, 128), jnp.float32),
        pltpu.VMEM((_BPW, 2 * _EMB), jnp.float32),
        pltpu.SemaphoreType.DMA,
    ],
    mesh=plsc.VectorSubcoreMesh(core_axis_name="c", subcore_axis_name="s"),
    compiler_params=pltpu.CompilerParams(use_tc_tiling_on_sc=False,
                                         needs_layout_passes=False),
)
def _sc_id_merge(ids_hbm, tid_hbm, outa_hbm, out_hbm,
                 ids_v, idrows_v, outaw_v, outw_v, sem_id):
    c = lax.axis_index("c")
    s = lax.axis_index("s")
    w = s * _NC + c

    pltpu.sync_copy(ids_hbm.at[pl.ds(w * _BPW, _BPW)], ids_v)
    for q in range(4):
        pltpu.async_copy(tid_hbm.at[ids_v.at[pl.ds(q * 128, 128)]],
                         idrows_v.at[pl.ds(q * 128, 128), :], sem_id)
    pltpu.sync_copy(outa_hbm.at[pl.ds(w * (_BPW // 2), _BPW // 2), :], outaw_v)
    pltpu.make_async_copy(tid_hbm.at[pl.ds(0, _BPW), :], idrows_v, sem_id).wait()

    def group_body(g, carry):
        for j in range(16):
            b = g * 16 + j
            arow = g * 8 + j // 2
            acol = 64 * (j % 2)
            outw_v[b, pl.ds(0, 16)] = idrows_v[b, pl.ds(0, 16)]
            outw_v[b, pl.ds(16, 16)] = idrows_v[b, pl.ds(16, 16)]
            outw_v[b, pl.ds(32, 16)] = outaw_v[arow, pl.ds(acol + 32, 16)]
            outw_v[b, pl.ds(48, 16)] = outaw_v[arow, pl.ds(acol + 48, 16)]
        return carry

    lax.fori_loop(0, _BPW // 16, group_body, 0)
    pltpu.sync_copy(outw_v, out_hbm.at[pl.ds(w * _BPW, _BPW), pl.ds(0, 2 * _EMB)])


@jax.jit
def kernel(item_ids, title_tokens, table_id, table_text):
    ids = item_ids.astype(jnp.int32)
    tok = title_tokens.astype(jnp.int32).reshape(_B * _L // 128, 128)
    outa = _sc_item_model(tok, table_text)
    out = _sc_id_merge(ids, table_id, outa)
    return out[:, :2 * _EMB]
